# Initial kernel scaffold; baseline (speedup 1.0000x reference)
import jax
import jax.numpy as jnp
from jax.experimental import pallas as pl


def kernel(x, edge_index, edge_vals):
    raise NotImplementedError("write your pallas kernel here")


# speedup vs baseline: 4.5558x; 4.5558x over previous
import functools

import jax
import jax.numpy as jnp
from jax import lax
from jax.experimental import pallas as pl
from jax.experimental.pallas import tpu as pltpu
from jax.experimental.pallas import tpu_sc as plsc

N = 10000
E = 320000
D = 128
DH = D // 2
NC = 2
NS = 16
LANES = 16
EPT = E // NS
K = 80
CH = EPT // K
RB = 624
ZR = 208
TAIL = N - NS * RB


def _sc_body(x_hbm, col_hbm, row_hbm, vals_hbm, out_hbm,
             col_buf, row_buf, vals_buf, gbuf, zbuf, accum, sem):
    cid = lax.axis_index("c")
    sid = lax.axis_index("s")

    def zrow(r, _):
        for f in range(DH // LANES):
            zbuf[r, pl.ds(f * LANES, LANES)] = jnp.zeros((LANES,), jnp.float32)
        return 0
    lax.fori_loop(0, ZR, zrow, 0)
    for z in range(RB // ZR):
        pltpu.sync_copy(zbuf, accum.at[pl.ds(sid * RB + z * ZR, ZR)])

    @pl.when(sid == NS - 1)
    def _zero_tail():
        pltpu.sync_copy(zbuf.at[pl.ds(0, TAIL)], accum.at[pl.ds(NS * RB, TAIL)])
    plsc.subcore_barrier()

    pltpu.sync_copy(col_hbm.at[sid], col_buf)
    pltpu.sync_copy(row_hbm.at[sid], row_buf)
    pltpu.sync_copy(vals_hbm.at[sid], vals_buf)

    off = cid * N

    def adj(r, _):
        for f in range(K // LANES):
            sl = pl.ds(f * LANES, LANES)
            col_buf[r, sl] = col_buf[r, sl] + off
        return 0
    lax.fori_loop(0, CH, adj, 0)

    def chunk(j, _):
        pltpu.async_copy(x_hbm.at[col_buf.at[j]], gbuf, sem).wait()

        def scale(g, _):
            vv = vals_buf[j, pl.ds(g * LANES, LANES)]
            for i in range(LANES):
                v = vv[i]
                e = g * LANES + i
                for f in range(DH // LANES):
                    sl = pl.ds(f * LANES, LANES)
                    gbuf[e, sl] = gbuf[e, sl] * v
            return 0
        lax.fori_loop(0, K // LANES, scale, 0)

        pltpu.sync_copy(gbuf, accum.at[row_buf.at[j]], add=True)
        return 0
    lax.fori_loop(0, CH, chunk, 0)

    plsc.subcore_barrier()
    pltpu.sync_copy(accum.at[pl.ds(sid * RB, RB)],
                    out_hbm.at[cid, pl.ds(sid * RB, RB)])

    @pl.when(sid == NS - 1)
    def _flush_tail():
        pltpu.sync_copy(accum.at[pl.ds(NS * RB, TAIL)],
                        out_hbm.at[cid, pl.ds(NS * RB, TAIL)])


_spmm_sc = functools.partial(
    pl.kernel,
    out_type=jax.ShapeDtypeStruct((NC, N, DH), jnp.float32),
    mesh=plsc.VectorSubcoreMesh(core_axis_name="c", subcore_axis_name="s"),
    compiler_params=pltpu.CompilerParams(use_tc_tiling_on_sc=False),
    scratch_types=[
        pltpu.VMEM((CH, K), jnp.int32),
        pltpu.VMEM((CH, K), jnp.int32),
        pltpu.VMEM((CH, K), jnp.float32),
        pltpu.VMEM((K, DH), jnp.float32),
        pltpu.VMEM((ZR, DH), jnp.float32),
        pltpu.VMEM_SHARED((N, DH), jnp.float32),
        pltpu.SemaphoreType.DMA,
    ],
)(_sc_body)


def _combine_body(p_ref, o_ref):
    o_ref[...] = jnp.concatenate([p_ref[0], p_ref[1]], axis=-1)


def _combine(partials):
    return pl.pallas_call(
        _combine_body,
        grid=(10,),
        in_specs=[pl.BlockSpec((NC, N // 10, DH), lambda i: (0, i, 0))],
        out_specs=pl.BlockSpec((N // 10, D), lambda i: (i, 0)),
        out_shape=jax.ShapeDtypeStruct((N, D), jnp.float32),
    )(partials)


def kernel(x, edge_index, edge_vals):
    xcat = jnp.concatenate([x[:, :DH], x[:, DH:]], axis=0)
    col_r = edge_index[1].reshape(NS, CH, K)
    row_r = edge_index[0].reshape(NS, CH, K)
    vals_r = edge_vals.reshape(NS, CH, K)
    partials = _spmm_sc(xcat, col_r, row_r, vals_r)
    return _combine(partials)


# speedup vs baseline: 8.3233x; 1.8270x over previous
import functools

import jax
import jax.numpy as jnp
from jax import lax
from jax.experimental import pallas as pl
from jax.experimental.pallas import tpu as pltpu
from jax.experimental.pallas import tpu_sc as plsc

N = 10000
E = 320000
D = 128
DH = D // 2
NC = 2
NS = 16
LANES = 16
EPT = E // NS
K = 80
CH = EPT // K
NB = 3
RB = 624
ZR = 208
TAIL = N - NS * RB


def _sc_body(x_hbm, col_hbm, row_hbm, vals_hbm, out_hbm,
             col_buf, row_buf, vals_buf, zbuf, accum,
             g0, g1, g2,
             sg0, sg1, sg2,
             ss0, ss1, ss2):
    cid = lax.axis_index("c")
    sid = lax.axis_index("s")
    bufs = (g0, g1, g2)
    sems_g = (sg0, sg1, sg2)
    sems_s = (ss0, ss1, ss2)

    def zrow(r, _):
        for f in range(DH // LANES):
            zbuf[r, pl.ds(f * LANES, LANES)] = jnp.zeros((LANES,), jnp.float32)
        return 0
    lax.fori_loop(0, ZR, zrow, 0)
    for z in range(RB // ZR):
        pltpu.sync_copy(zbuf, accum.at[pl.ds(sid * RB + z * ZR, ZR)])

    @pl.when(sid == NS - 1)
    def _zero_tail():
        pltpu.sync_copy(zbuf.at[pl.ds(0, TAIL)], accum.at[pl.ds(NS * RB, TAIL)])
    plsc.subcore_barrier()

    pltpu.sync_copy(col_hbm.at[sid], col_buf)
    pltpu.sync_copy(row_hbm.at[sid], row_buf)
    pltpu.sync_copy(vals_hbm.at[sid], vals_buf)

    off = cid * N

    def adj(r, _):
        for f in range(K // LANES):
            sl = pl.ds(f * LANES, LANES)
            col_buf[r, sl] = col_buf[r, sl] + off
        return 0
    lax.fori_loop(0, CH, adj, 0)

    def issue_gather(j, b):
        pltpu.async_copy(x_hbm.at[col_buf.at[j]], bufs[b], sems_g[b])

    def wait_gather(b):
        pltpu.make_async_copy(x_hbm.at[col_buf.at[0]], bufs[b],
                              sems_g[b]).wait()

    def issue_scatter(j, b):
        pltpu.async_copy(bufs[b], accum.at[row_buf.at[j]], sems_s[b], add=True)

    def wait_scatter(b):
        pltpu.make_async_copy(bufs[b], accum.at[row_buf.at[0]],
                              sems_s[b]).wait()

    def scale(j, b):
        gb = bufs[b]

        def body(g, _):
            vv = vals_buf[j, pl.ds(g * LANES, LANES)]
            for i in range(LANES):
                v = vv[i]
                e = g * LANES + i
                for f in range(DH // LANES):
                    sl = pl.ds(f * LANES, LANES)
                    gb[e, sl] = gb[e, sl] * v
            return 0
        lax.fori_loop(0, K // LANES, body, 0)

    issue_gather(0, 0)
    issue_gather(1, 1)
    for b in range(NB):
        j = b
        wait_gather(b)
        scale(j, b)
        issue_scatter(j, b)
        if j >= 1:
            wait_scatter((b - 1) % NB)
        issue_gather(j + 2, (b + 2) % NB)

    def outer(p, _):
        for b in range(NB):
            j = p * NB + b
            wait_gather(b)
            scale(j, b)
            issue_scatter(j, b)
            wait_scatter((b - 1) % NB)

            @pl.when(j + 2 < CH)
            def _():
                issue_gather(j + 2, (b + 2) % NB)
        return 0
    lax.fori_loop(1, (CH - 1) // NB, outer, 0)

    jl = CH - 1
    bl = jl % NB
    wait_gather(bl)
    scale(jl, bl)
    issue_scatter(jl, bl)
    wait_scatter((bl - 1) % NB)
    wait_scatter(bl)

    plsc.subcore_barrier()
    pltpu.sync_copy(accum.at[pl.ds(sid * RB, RB)],
                    out_hbm.at[cid, pl.ds(sid * RB, RB)])

    @pl.when(sid == NS - 1)
    def _flush_tail():
        pltpu.sync_copy(accum.at[pl.ds(NS * RB, TAIL)],
                        out_hbm.at[cid, pl.ds(NS * RB, TAIL)])


_spmm_sc = functools.partial(
    pl.kernel,
    out_type=jax.ShapeDtypeStruct((NC, N, DH), jnp.float32),
    mesh=plsc.VectorSubcoreMesh(core_axis_name="c", subcore_axis_name="s"),
    compiler_params=pltpu.CompilerParams(use_tc_tiling_on_sc=False),
    scratch_types=(
        [
            pltpu.VMEM((CH, K), jnp.int32),
            pltpu.VMEM((CH, K), jnp.int32),
            pltpu.VMEM((CH, K), jnp.float32),
            pltpu.VMEM((ZR, DH), jnp.float32),
            pltpu.VMEM_SHARED((N, DH), jnp.float32),
        ]
        + [pltpu.VMEM((K, DH), jnp.float32) for _ in range(NB)]
        + [pltpu.SemaphoreType.DMA for _ in range(2 * NB)]
    ),
)(_sc_body)


def _combine_body(p_ref, o_ref):
    o_ref[...] = jnp.concatenate([p_ref[0], p_ref[1]], axis=-1)


def _combine(partials):
    return pl.pallas_call(
        _combine_body,
        grid=(10,),
        in_specs=[pl.BlockSpec((NC, N // 10, DH), lambda i: (0, i, 0))],
        out_specs=pl.BlockSpec((N // 10, D), lambda i: (i, 0)),
        out_shape=jax.ShapeDtypeStruct((N, D), jnp.float32),
    )(partials)


def kernel(x, edge_index, edge_vals):
    xcat = jnp.concatenate([x[:, :DH], x[:, DH:]], axis=0)
    col_r = edge_index[1].reshape(NS, CH, K)
    row_r = edge_index[0].reshape(NS, CH, K)
    vals_r = edge_vals.reshape(NS, CH, K)
    partials = _spmm_sc(xcat, col_r, row_r, vals_r)
    return _combine(partials)


# speedup vs baseline: 9.0537x; 1.0878x over previous
import functools

import jax
import jax.numpy as jnp
from jax import lax
from jax.experimental import pallas as pl
from jax.experimental.pallas import tpu as pltpu
from jax.experimental.pallas import tpu_sc as plsc

N = 10000
E = 320000
D = 128
DH = D // 2
NC = 2
NS = 16
LANES = 16
EPT = E // NS
K = 80
CH = EPT // K
NB = 3
RB = 624
ZR = 208
TAIL = N - NS * RB


def _sc_body(x_hbm, col_hbm, row_hbm, vals_hbm, out_hbm,
             col_buf, row_buf, vals_buf, zbuf, accum,
             g0, g1, g2,
             sg0, sg1, sg2,
             ss0, ss1, ss2):
    cid = lax.axis_index("c")
    sid = lax.axis_index("s")
    bufs = (g0, g1, g2)
    sems_g = (sg0, sg1, sg2)
    sems_s = (ss0, ss1, ss2)

    def zrow(r, _):
        for f in range(DH // LANES):
            zbuf[r, pl.ds(f * LANES, LANES)] = jnp.zeros((LANES,), jnp.float32)
        return 0
    lax.fori_loop(0, ZR, zrow, 0)
    for z in range(RB // ZR):
        pltpu.sync_copy(zbuf, accum.at[pl.ds(sid * RB + z * ZR, ZR)])

    @pl.when(sid == NS - 1)
    def _zero_tail():
        pltpu.sync_copy(zbuf.at[pl.ds(0, TAIL)], accum.at[pl.ds(NS * RB, TAIL)])
    plsc.subcore_barrier()

    pltpu.sync_copy(col_hbm.at[sid], col_buf)
    pltpu.sync_copy(row_hbm.at[sid], row_buf)
    pltpu.sync_copy(vals_hbm.at[sid], vals_buf)

    off = cid * N

    def adj(r, _):
        for f in range(K // LANES):
            sl = pl.ds(f * LANES, LANES)
            col_buf[r, sl] = col_buf[r, sl] + off
        return 0
    lax.fori_loop(0, CH, adj, 0)

    def issue_gather(j, b):
        pltpu.async_copy(x_hbm.at[col_buf.at[j]], bufs[b], sems_g[b])

    def wait_gather(b):
        pltpu.make_async_copy(x_hbm.at[col_buf.at[0]], bufs[b],
                              sems_g[b]).wait()

    def issue_scatter(j, b):
        pltpu.async_copy(bufs[b], accum.at[row_buf.at[j]], sems_s[b], add=True)

    def wait_scatter(b):
        pltpu.make_async_copy(bufs[b], accum.at[row_buf.at[0]],
                              sems_s[b]).wait()

    def scale(j, b):
        gb = bufs[b]

        def body(g, _):
            vv = vals_buf[j, pl.ds(g * LANES, LANES)]
            for i in range(LANES):
                v = vv[i]
                e = g * LANES + i
                for f in range(DH // LANES):
                    sl = pl.ds(f * LANES, LANES)
                    gb[e, sl] = gb[e, sl] * v
            return 0
        lax.fori_loop(0, K // LANES, body, 0)

    issue_gather(0, 0)
    issue_gather(1, 1)
    for b in range(NB):
        j = b
        wait_gather(b)
        scale(j, b)
        issue_scatter(j, b)
        if j >= 1:
            wait_scatter((b - 1) % NB)
        issue_gather(j + NB - 1, (b + NB - 1) % NB)

    def outer(p, _):
        for b in range(NB):
            j = p * NB + b
            wait_gather(b)
            scale(j, b)
            issue_scatter(j, b)
            wait_scatter((b - 1) % NB)

            @pl.when(j + NB - 1 < CH)
            def _():
                issue_gather(j + NB - 1, (b + NB - 1) % NB)
        return 0
    lax.fori_loop(1, 1 + (CH - NB) // NB, outer, 0)

    for jl in range(NB + NB * ((CH - NB) // NB), CH):
        bl = jl % NB
        wait_gather(bl)
        scale(jl, bl)
        issue_scatter(jl, bl)
        wait_scatter((bl - 1) % NB)
    wait_scatter((CH - 1) % NB)

    plsc.subcore_barrier()
    pltpu.sync_copy(accum.at[pl.ds(sid * RB, RB)],
                    out_hbm.at[pl.ds(sid * RB, RB), pl.ds(cid * DH, DH)])

    @pl.when(sid == NS - 1)
    def _flush_tail():
        pltpu.sync_copy(accum.at[pl.ds(NS * RB, TAIL)],
                        out_hbm.at[pl.ds(NS * RB, TAIL), pl.ds(cid * DH, DH)])


_spmm_sc = functools.partial(
    pl.kernel,
    out_type=jax.ShapeDtypeStruct((N, D), jnp.float32),
    mesh=plsc.VectorSubcoreMesh(core_axis_name="c", subcore_axis_name="s"),
    compiler_params=pltpu.CompilerParams(use_tc_tiling_on_sc=False),
    scratch_types=(
        [
            pltpu.VMEM((CH, K), jnp.int32),
            pltpu.VMEM((CH, K), jnp.int32),
            pltpu.VMEM((CH, K), jnp.float32),
            pltpu.VMEM((ZR, DH), jnp.float32),
            pltpu.VMEM_SHARED((N, DH), jnp.float32),
        ]
        + [pltpu.VMEM((K, DH), jnp.float32) for _ in range(NB)]
        + [pltpu.SemaphoreType.DMA for _ in range(2 * NB)]
    ),
)(_sc_body)


def kernel(x, edge_index, edge_vals):
    xcat = jnp.concatenate([x[:, :DH], x[:, DH:]], axis=0)
    col_r = edge_index[1].reshape(NS, CH, K)
    row_r = edge_index[0].reshape(NS, CH, K)
    vals_r = edge_vals.reshape(NS, CH, K)
    return _spmm_sc(xcat, col_r, row_r, vals_r)
